# baseline (speedup 1.0000x reference)
import functools

import jax
import jax.numpy as jnp
from jax import lax
from jax.experimental import pallas as pl
from jax.experimental.pallas import tpu as pltpu
from jax.experimental.pallas import tpu_sc as plsc

_DIM = 64
_SCALE = 8.0
_ROWS_BLOCK = 5000


def _transform_body(table_ref, out_ref):
    pos = lax.broadcasted_iota(jnp.int32, (_ROWS_BLOCK, _DIM), 1).astype(jnp.float32) + 1.0
    out_ref[...] = table_ref[...] * _SCALE + pos


def _transform(table):
    vocab = table.shape[0]
    return pl.pallas_call(
        _transform_body,
        grid=(vocab // _ROWS_BLOCK,),
        in_specs=[pl.BlockSpec((_ROWS_BLOCK, _DIM), lambda i: (i, 0))],
        out_specs=pl.BlockSpec((_ROWS_BLOCK, _DIM), lambda i: (i, 0)),
        out_shape=jax.ShapeDtypeStruct((vocab, _DIM), jnp.float32),
    )(table)


@functools.lru_cache(maxsize=None)
def _make_gather(batch, seq, vocab):
    info = plsc.get_sparse_core_info()
    nc, ns = info.num_cores, info.num_subcores
    nw = nc * ns
    bpw = batch // nw
    mesh = plsc.VectorSubcoreMesh(core_axis_name="c", subcore_axis_name="s")

    @functools.partial(
        pl.kernel,
        mesh=mesh,
        compiler_params=pltpu.CompilerParams(use_tc_tiling_on_sc=False),
        out_type=jax.ShapeDtypeStruct((1, batch, seq, _DIM), jnp.float32),
        scratch_types=[
            pltpu.VMEM((bpw, seq), jnp.int32),
            pltpu.VMEM((seq, _DIM), jnp.float32),
            pltpu.VMEM((seq, _DIM), jnp.float32),
            pltpu.SemaphoreType.DMA,
            pltpu.SemaphoreType.DMA,
            pltpu.SemaphoreType.DMA,
            pltpu.SemaphoreType.DMA,
        ],
    )
    def k(idx_hbm, table_hbm, out_hbm, idx_v, buf_a, buf_b, gs_a, gs_b, ws_a, ws_b):
        wid = lax.axis_index("s") * nc + lax.axis_index("c")
        b0 = wid * bpw
        pltpu.sync_copy(idx_hbm.at[wid], idx_v)

        def start_gather(j, buf, sem):
            pltpu.async_copy(table_hbm.at[idx_v.at[j]], buf, sem)

        def start_write(j, buf, sem):
            pltpu.async_copy(buf, out_hbm.at[0, b0 + j], sem)

        start_gather(0, buf_a, gs_a)
        start_gather(1, buf_b, gs_b)

        def body(j2, carry):
            j = 2 * j2
            pltpu.make_async_copy(table_hbm.at[idx_v.at[j]], buf_a, gs_a).wait()
            start_write(j, buf_a, ws_a)
            pltpu.make_async_copy(table_hbm.at[idx_v.at[j + 1]], buf_b, gs_b).wait()
            start_write(j + 1, buf_b, ws_b)

            @pl.when(j + 2 < bpw)
            def _():
                pltpu.make_async_copy(buf_a, out_hbm.at[0, b0 + j], ws_a).wait()
                start_gather(j + 2, buf_a, gs_a)

            @pl.when(j + 3 < bpw)
            def _():
                pltpu.make_async_copy(buf_b, out_hbm.at[0, b0 + j + 1], ws_b).wait()
                start_gather(j + 3, buf_b, gs_b)

            return carry

        lax.fori_loop(0, bpw // 2, body, 0)
        pltpu.make_async_copy(buf_a, out_hbm.at[0, b0 + bpw - 2], ws_a).wait()
        pltpu.make_async_copy(buf_b, out_hbm.at[0, b0 + bpw - 1], ws_b).wait()

    return k


def kernel(x, table):
    b, l = x.shape
    nw = plsc.get_sparse_core_info().num_cores * plsc.get_sparse_core_info().num_subcores
    idx = x.reshape(nw, b // nw, l).astype(jnp.int32)
    table2 = _transform(table)
    return _make_gather(b, l, table.shape[0])(idx, table2)


# speedup vs baseline: 1.2845x; 1.2845x over previous
import functools

import jax
import jax.numpy as jnp
from jax import lax
from jax.experimental import pallas as pl
from jax.experimental.pallas import tpu as pltpu
from jax.experimental.pallas import tpu_sc as plsc

_DIM = 64
_SCALE = 8.0
_COLS_BLOCK = 4096


def _transform_body(tt_ref, out_ref):
    pos = lax.broadcasted_iota(jnp.int32, (_COLS_BLOCK, _DIM), 1).astype(jnp.float32) + 1.0
    out_ref[:, : _DIM] = tt_ref[...].T * _SCALE + pos


def _transform(table_t):
    vocab = table_t.shape[1]
    return pl.pallas_call(
        _transform_body,
        grid=((vocab + _COLS_BLOCK - 1) // _COLS_BLOCK,),
        in_specs=[pl.BlockSpec((_DIM, _COLS_BLOCK), lambda i: (0, i))],
        out_specs=pl.BlockSpec((_COLS_BLOCK, 2 * _DIM), lambda i: (i, 0)),
        out_shape=jax.ShapeDtypeStruct((vocab, 2 * _DIM), jnp.float32),
    )(table_t)


@functools.lru_cache(maxsize=None)
def _make_gather(batch, seq, vocab):
    info = plsc.get_sparse_core_info()
    nc, ns = info.num_cores, info.num_subcores
    nw = nc * ns
    bpw = batch // nw
    mesh = plsc.VectorSubcoreMesh(core_axis_name="c", subcore_axis_name="s")

    @functools.partial(
        pl.kernel,
        mesh=mesh,
        compiler_params=pltpu.CompilerParams(use_tc_tiling_on_sc=False),
        out_type=jax.ShapeDtypeStruct((1, batch, seq, _DIM), jnp.float32),
        scratch_types=[
            pltpu.VMEM((bpw, seq), jnp.int32),
            pltpu.VMEM((seq, _DIM), jnp.float32),
            pltpu.VMEM((seq, _DIM), jnp.float32),
            pltpu.SemaphoreType.DMA,
            pltpu.SemaphoreType.DMA,
            pltpu.SemaphoreType.DMA,
            pltpu.SemaphoreType.DMA,
        ],
    )
    def k(idx_hbm, table_hbm, out_hbm, idx_v, buf_a, buf_b, gs_a, gs_b, ws_a, ws_b):
        wid = lax.axis_index("s") * nc + lax.axis_index("c")
        b0 = wid * bpw
        pltpu.sync_copy(idx_hbm.at[wid], idx_v)

        def start_gather(j, buf, sem):
            pltpu.async_copy(table_hbm.at[idx_v.at[j]], buf, sem)

        def start_write(j, buf, sem):
            pltpu.async_copy(buf, out_hbm.at[0, b0 + j], sem)

        start_gather(0, buf_a, gs_a)
        start_gather(1, buf_b, gs_b)

        def body(j2, carry):
            j = 2 * j2
            pltpu.make_async_copy(table_hbm.at[idx_v.at[j]], buf_a, gs_a).wait()
            start_write(j, buf_a, ws_a)
            pltpu.make_async_copy(table_hbm.at[idx_v.at[j + 1]], buf_b, gs_b).wait()
            start_write(j + 1, buf_b, ws_b)

            @pl.when(j + 2 < bpw)
            def _():
                pltpu.make_async_copy(buf_a, out_hbm.at[0, b0 + j], ws_a).wait()
                start_gather(j + 2, buf_a, gs_a)

            @pl.when(j + 3 < bpw)
            def _():
                pltpu.make_async_copy(buf_b, out_hbm.at[0, b0 + j + 1], ws_b).wait()
                start_gather(j + 3, buf_b, gs_b)

            return carry

        lax.fori_loop(0, bpw // 2, body, 0)
        pltpu.make_async_copy(buf_a, out_hbm.at[0, b0 + bpw - 2], ws_a).wait()
        pltpu.make_async_copy(buf_b, out_hbm.at[0, b0 + bpw - 1], ws_b).wait()

    return k


def kernel(x, table):
    b, l = x.shape
    nw = plsc.get_sparse_core_info().num_cores * plsc.get_sparse_core_info().num_subcores
    idx = (x.astype(jnp.int32) * 2).reshape(nw, b // nw, l)
    table2 = _transform(table.T).reshape(2 * table.shape[0], _DIM)
    return _make_gather(b, l, 2 * table.shape[0])(idx, table2)
